# Initial kernel scaffold; baseline (speedup 1.0000x reference)
import jax
import jax.numpy as jnp
from jax.experimental import pallas as pl


def kernel(output, y):
    raise NotImplementedError("write your pallas kernel here")


# speedup vs baseline: 2.1077x; 2.1077x over previous
import jax
import jax.numpy as jnp
from jax.experimental import pallas as pl
from jax.experimental.pallas import tpu as pltpu

B = 16384
C = 1000
MARGIN = 1.0
BM = 512
NB = B // BM


def _hinge_block(x_ref, y_ref, out_ref):
    i = pl.program_id(0)
    x = x_ref[...]
    yv = y_ref[...]
    col = jax.lax.broadcasted_iota(jnp.int32, x.shape, 1)
    is_tgt = col == yv
    g = jnp.sum(jnp.where(is_tgt, x, 0.0), axis=1, keepdims=True)
    loss = x - g + MARGIN
    keep = jnp.logical_and(jnp.logical_not(is_tgt), col < C)
    loss = jnp.where(keep, loss, 0.0)
    partial = jnp.sum(loss * loss).reshape(1, 1)

    @pl.when(i == 0)
    def _init():
        out_ref[...] = jnp.zeros((1, 1), jnp.float32)

    out_ref[...] += partial

    @pl.when(i == NB - 1)
    def _finish():
        out_ref[...] = out_ref[...] / B


def kernel(output, y):
    y2 = y.reshape(B, 1)
    total = pl.pallas_call(
        _hinge_block,
        grid=(NB,),
        in_specs=[
            pl.BlockSpec((BM, C), lambda i: (i, 0)),
            pl.BlockSpec((BM, 1), lambda i: (i, 0)),
        ],
        out_specs=pl.BlockSpec((1, 1), lambda i: (0, 0)),
        out_shape=jax.ShapeDtypeStruct((1, 1), jnp.float32),
        compiler_params=pltpu.CompilerParams(
            dimension_semantics=("arbitrary",),
        ),
    )(output, y2)
    return total[0, 0]


# speedup vs baseline: 2.3034x; 1.0928x over previous
import jax
import jax.numpy as jnp
from jax.experimental import pallas as pl
from jax.experimental.pallas import tpu as pltpu

B = 16384
C = 1000
MARGIN = 1.0
BM = 512
NB = B // BM


def _hinge_block(x_ref, y_ref, out_ref):
    i = pl.program_id(0)
    x = x_ref[...]
    yv = y_ref[...].reshape(BM, 1)
    col = jax.lax.broadcasted_iota(jnp.int32, x.shape, 1)
    is_tgt = col == yv
    g = jnp.sum(jnp.where(is_tgt, x, 0.0), axis=1, keepdims=True)
    loss = x - g + MARGIN
    keep = jnp.logical_and(jnp.logical_not(is_tgt), col < C)
    loss = jnp.where(keep, loss, 0.0)
    partial = jnp.sum(loss * loss).reshape(1, 1)

    @pl.when(i == 0)
    def _init():
        out_ref[...] = jnp.zeros((1, 1), jnp.float32)

    out_ref[...] += partial

    @pl.when(i == NB - 1)
    def _finish():
        out_ref[...] = out_ref[...] / B


def kernel(output, y):
    total = pl.pallas_call(
        _hinge_block,
        grid=(NB,),
        in_specs=[
            pl.BlockSpec((BM, C), lambda i: (i, 0)),
            pl.BlockSpec((BM,), lambda i: (i,)),
        ],
        out_specs=pl.BlockSpec((1, 1), lambda i: (0, 0)),
        out_shape=jax.ShapeDtypeStruct((1, 1), jnp.float32),
        compiler_params=pltpu.CompilerParams(
            dimension_semantics=("arbitrary",),
        ),
    )(output, y)
    return total[0, 0]


# speedup vs baseline: 6.4214x; 2.7878x over previous
import jax
import jax.numpy as jnp
from jax.experimental import pallas as pl
from jax.experimental.pallas import tpu as pltpu

B = 16384
C = 1000
MARGIN = 1.0
BN = 512
NB = B // BN


def _hinge_block(x_ref, y_ref, out_ref):
    i = pl.program_id(0)
    x = x_ref[...]
    yv = y_ref[...].reshape(1, BN)
    rows = jax.lax.broadcasted_iota(jnp.int32, (C, BN), 0)
    g = jnp.sum(jnp.where(rows == yv, x, 0.0), axis=0, keepdims=True)
    s1 = jnp.sum(x, axis=0, keepdims=True)
    s2 = jnp.sum(x * x, axis=0, keepdims=True)
    omg = 1.0 - g
    row_tot = s2 + 2.0 * omg * s1 + C * (omg * omg) - 1.0
    partial = jnp.sum(row_tot).reshape(1, 1)

    @pl.when(i == 0)
    def _init():
        out_ref[...] = jnp.zeros((1, 1), jnp.float32)

    out_ref[...] += partial

    @pl.when(i == NB - 1)
    def _finish():
        out_ref[...] = out_ref[...] / B


def kernel(output, y):
    xt = output.T
    total = pl.pallas_call(
        _hinge_block,
        grid=(NB,),
        in_specs=[
            pl.BlockSpec((C, BN), lambda i: (0, i)),
            pl.BlockSpec((BN,), lambda i: (i,)),
        ],
        out_specs=pl.BlockSpec((1, 1), lambda i: (0, 0)),
        out_shape=jax.ShapeDtypeStruct((1, 1), jnp.float32),
        compiler_params=pltpu.CompilerParams(
            dimension_semantics=("arbitrary",),
        ),
    )(xt, y)
    return total[0, 0]


# speedup vs baseline: 7.9491x; 1.2379x over previous
import jax
import jax.numpy as jnp
from jax.experimental import pallas as pl
from jax.experimental.pallas import tpu as pltpu

B = 16384
C = 1000
MARGIN = 1.0
BN = 1024
NB = B // BN


def _hinge_block(x_ref, y_ref, out_ref):
    i = pl.program_id(0)
    x = x_ref[...]
    yv = y_ref[...].reshape(1, BN)
    rows = jax.lax.broadcasted_iota(jnp.int32, (C, BN), 0)
    g = jnp.sum(jnp.where(rows == yv, x, 0.0), axis=0, keepdims=True)
    s1 = jnp.sum(x, axis=0, keepdims=True)
    s2 = jnp.sum(x * x, axis=0, keepdims=True)
    omg = 1.0 - g
    row_tot = s2 + 2.0 * omg * s1 + C * (omg * omg) - 1.0
    partial = jnp.sum(row_tot).reshape(1, 1)

    @pl.when(i == 0)
    def _init():
        out_ref[...] = jnp.zeros((1, 1), jnp.float32)

    out_ref[...] += partial

    @pl.when(i == NB - 1)
    def _finish():
        out_ref[...] = out_ref[...] / B


def kernel(output, y):
    xt = output.T
    total = pl.pallas_call(
        _hinge_block,
        grid=(NB,),
        in_specs=[
            pl.BlockSpec((C, BN), lambda i: (0, i)),
            pl.BlockSpec((BN,), lambda i: (i,)),
        ],
        out_specs=pl.BlockSpec((1, 1), lambda i: (0, 0)),
        out_shape=jax.ShapeDtypeStruct((1, 1), jnp.float32),
        compiler_params=pltpu.CompilerParams(
            dimension_semantics=("arbitrary",),
        ),
    )(xt, y)
    return total[0, 0]


# speedup vs baseline: 9.0418x; 1.1375x over previous
import jax
import jax.numpy as jnp
from jax.experimental import pallas as pl
from jax.experimental.pallas import tpu as pltpu

B = 16384
C = 1000
MARGIN = 1.0
BN = 2048
NB = B // BN


def _hinge_block(x_ref, y_ref, out_ref):
    i = pl.program_id(0)
    x = x_ref[...]
    yv = y_ref[...].reshape(1, BN)
    rows = jax.lax.broadcasted_iota(jnp.int32, (C, BN), 0)
    g = jnp.sum(jnp.where(rows == yv, x, 0.0), axis=0, keepdims=True)
    s1 = jnp.sum(x, axis=0, keepdims=True)
    s2 = jnp.sum(x * x, axis=0, keepdims=True)
    omg = 1.0 - g
    row_tot = s2 + 2.0 * omg * s1 + C * (omg * omg) - 1.0
    partial = jnp.sum(row_tot).reshape(1, 1)

    @pl.when(i == 0)
    def _init():
        out_ref[...] = jnp.zeros((1, 1), jnp.float32)

    out_ref[...] += partial

    @pl.when(i == NB - 1)
    def _finish():
        out_ref[...] = out_ref[...] / B


def kernel(output, y):
    xt = output.T
    total = pl.pallas_call(
        _hinge_block,
        grid=(NB,),
        in_specs=[
            pl.BlockSpec((C, BN), lambda i: (0, i)),
            pl.BlockSpec((BN,), lambda i: (i,)),
        ],
        out_specs=pl.BlockSpec((1, 1), lambda i: (0, 0)),
        out_shape=jax.ShapeDtypeStruct((1, 1), jnp.float32),
        compiler_params=pltpu.CompilerParams(
            dimension_semantics=("arbitrary",),
        ),
    )(xt, y)
    return total[0, 0]
